# Initial kernel scaffold; baseline (speedup 1.0000x reference)
import jax
import jax.numpy as jnp
from jax.experimental import pallas as pl


def kernel(query, G, gamma):
    raise NotImplementedError("write your pallas kernel here")


# speedup vs baseline: 2.0309x; 2.0309x over previous
import functools

import jax
import jax.numpy as jnp
from jax.experimental import pallas as pl
from jax.experimental.pallas import tpu as pltpu

_EPS = 1e-12

_B = 1024
_D = 16
_N = 100000
_BLK = 2000
_NBLK = _N // _BLK


def _attn_body(q_ref, g_ref, out_ref, acc_ref, den_ref):
    i = pl.program_id(0)

    @pl.when(i == 0)
    def _init():
        acc_ref[...] = jnp.zeros_like(acc_ref)
        den_ref[...] = jnp.zeros_like(den_ref)

    q = q_ref[...]
    qn = q / jnp.maximum(
        jnp.sqrt(jnp.sum(q * q, axis=1, keepdims=True)), _EPS
    )
    g = g_ref[...]
    gn = g / jnp.maximum(
        jnp.sqrt(jnp.sum(g * g, axis=1, keepdims=True)), _EPS
    )
    sim = jax.lax.dot_general(
        qn, gn,
        (((1,), (1,)), ((), ())),
        preferred_element_type=jnp.float32,
    )
    e = jnp.exp(sim)
    acc_ref[...] += jnp.dot(e, g, preferred_element_type=jnp.float32)
    den_ref[...] += jnp.sum(e, axis=1, keepdims=True)

    @pl.when(i == _NBLK - 1)
    def _fin():
        out_ref[...] = acc_ref[...] / den_ref[...]


def _gamma_body(gamma_ref, out_ref):
    g = gamma_ref[...]
    m = jnp.max(g)
    e = jnp.exp(g - m)
    out_ref[...] = e / jnp.sum(e)


@jax.jit
def kernel(query, G, gamma):
    g_out = pl.pallas_call(
        _attn_body,
        grid=(_NBLK,),
        in_specs=[
            pl.BlockSpec((_B, _D), lambda i: (0, 0)),
            pl.BlockSpec((_BLK, _D), lambda i: (i, 0)),
        ],
        out_specs=pl.BlockSpec((_B, _D), lambda i: (0, 0)),
        out_shape=jax.ShapeDtypeStruct((_B, _D), jnp.float32),
        scratch_shapes=[
            pltpu.VMEM((_B, _D), jnp.float32),
            pltpu.VMEM((_B, 1), jnp.float32),
        ],
    )(query, G)

    gamma2d = gamma.reshape(100, 1000)
    ng = pl.pallas_call(
        _gamma_body,
        out_shape=jax.ShapeDtypeStruct((100, 1000), jnp.float32),
    )(gamma2d)
    return g_out, ng.reshape(_N)


# speedup vs baseline: 2.2525x; 1.1091x over previous
import jax
import jax.numpy as jnp
from jax.experimental import pallas as pl
from jax.experimental.pallas import tpu as pltpu

_EPS = 1e-12
_LOG2E = 1.4426950408889634

_B = 1024
_D = 16
_N = 100000
_BLK = 2000
_NBLK = _N // _BLK


def _attn_body(q_ref, g_ref, out_ref, qn_ref, acc_ref):
    i = pl.program_id(0)

    @pl.when(i == 0)
    def _init():
        q = q_ref[...]
        qn_ref[...] = q / jnp.maximum(
            jnp.sqrt(jnp.sum(q * q, axis=1, keepdims=True)), _EPS
        )
        acc_ref[...] = jnp.zeros_like(acc_ref)

    g = g_ref[...]
    norm = jnp.sqrt(jnp.sum(g * g, axis=1, keepdims=True))
    scale = _LOG2E / jnp.maximum(norm, _EPS)
    gs = g * scale
    sim2 = jax.lax.dot_general(
        qn_ref[...], gs,
        (((1,), (1,)), ((), ())),
        preferred_element_type=jnp.float32,
    )
    e = jnp.exp2(sim2)
    gx = jnp.concatenate(
        [g, jnp.ones((_BLK, 1), jnp.float32)], axis=1
    )
    acc_ref[...] += jnp.dot(e, gx, preferred_element_type=jnp.float32)

    @pl.when(i == _NBLK - 1)
    def _fin():
        out_ref[...] = acc_ref[:, :_D] / acc_ref[:, _D:]


def _gamma_body(gamma_ref, out_ref):
    g = gamma_ref[...]
    m = jnp.max(g)
    e = jnp.exp(g - m)
    out_ref[...] = e / jnp.sum(e)


@jax.jit
def kernel(query, G, gamma):
    g_out = pl.pallas_call(
        _attn_body,
        grid=(_NBLK,),
        in_specs=[
            pl.BlockSpec((_B, _D), lambda i: (0, 0)),
            pl.BlockSpec((_BLK, _D), lambda i: (i, 0)),
        ],
        out_specs=pl.BlockSpec((_B, _D), lambda i: (0, 0)),
        out_shape=jax.ShapeDtypeStruct((_B, _D), jnp.float32),
        scratch_shapes=[
            pltpu.VMEM((_B, _D), jnp.float32),
            pltpu.VMEM((_B, _D + 1), jnp.float32),
        ],
    )(query, G)

    gamma2d = gamma.reshape(100, 1000)
    ng = pl.pallas_call(
        _gamma_body,
        out_shape=jax.ShapeDtypeStruct((100, 1000), jnp.float32),
    )(gamma2d)
    return g_out, ng.reshape(_N)


# speedup vs baseline: 2.4659x; 1.0947x over previous
import jax
import jax.numpy as jnp
from jax.experimental import pallas as pl
from jax.experimental.pallas import tpu as pltpu

_EPS = 1e-12
_LOG2E = 1.4426950408889634

_B = 1024
_D = 16
_N = 100000
_BLK = 5000
_NCORE = 2
_NIN = _N // _BLK // _NCORE


def _attn_body(q_ref, gam_ref, g_ref, out_ref, ng_ref, qn_ref, acc_ref):
    i = pl.program_id(1)

    @pl.when(i == 0)
    def _init():
        q = q_ref[...]
        qn_ref[...] = (
            q / jnp.maximum(
                jnp.sqrt(jnp.sum(q * q, axis=1, keepdims=True)), _EPS
            )
        ).astype(jnp.bfloat16)
        acc_ref[...] = jnp.zeros_like(acc_ref)
        gam = gam_ref[...]
        eg = jnp.exp(gam - jnp.max(gam))
        ng_ref[...] = eg / jnp.sum(eg)

    g = g_ref[...]
    sumsq = jnp.sum(g * g, axis=1, keepdims=True)
    scale = _LOG2E * jax.lax.rsqrt(jnp.maximum(sumsq, _EPS * _EPS))
    gs = (g * scale).astype(jnp.bfloat16)
    sim2 = jax.lax.dot_general(
        qn_ref[...], gs,
        (((1,), (1,)), ((), ())),
        preferred_element_type=jnp.float32,
    )
    e = jnp.exp2(sim2.astype(jnp.bfloat16))
    gx = jnp.concatenate(
        [g, jnp.ones((_BLK, 1), jnp.float32)], axis=1
    ).astype(jnp.bfloat16)
    acc_ref[...] += jnp.dot(e, gx, preferred_element_type=jnp.float32)

    @pl.when(i == _NIN - 1)
    def _fin():
        out_ref[...] = acc_ref[...][None]


@jax.jit
def kernel(query, G, gamma):
    partials, ng = pl.pallas_call(
        _attn_body,
        grid=(_NCORE, _NIN),
        in_specs=[
            pl.BlockSpec((_B, _D), lambda c, i: (0, 0)),
            pl.BlockSpec((100, 1000), lambda c, i: (0, 0)),
            pl.BlockSpec((_BLK, _D), lambda c, i: (c * _NIN + i, 0)),
        ],
        out_specs=[
            pl.BlockSpec((1, _B, _D + 1), lambda c, i: (c, 0, 0)),
            pl.BlockSpec((100, 1000), lambda c, i: (0, 0)),
        ],
        out_shape=[
            jax.ShapeDtypeStruct((_NCORE, _B, _D + 1), jnp.float32),
            jax.ShapeDtypeStruct((100, 1000), jnp.float32),
        ],
        scratch_shapes=[
            pltpu.VMEM((_B, _D), jnp.bfloat16),
            pltpu.VMEM((_B, _D + 1), jnp.float32),
        ],
        compiler_params=pltpu.CompilerParams(
            dimension_semantics=("parallel", "arbitrary"),
        ),
    )(query, gamma.reshape(100, 1000), G)
    tot = partials.sum(axis=0)
    g_out = tot[:, :_D] / tot[:, _D:]
    return g_out, ng.reshape(_N)


# speedup vs baseline: 2.9042x; 1.1777x over previous
import jax
import jax.numpy as jnp
from jax.experimental import pallas as pl
from jax.experimental.pallas import tpu as pltpu

_EPS = 1e-12
_LOG2E = 1.4426950408889634

_B = 1024
_D = 16
_N = 100000
_BLK = 5000
_NBLK = _N // _BLK
_F8 = jnp.float8_e4m3fn


def _attn_body(q_ref, gam_ref, g_ref, out_ref, ng_ref, qn_ref, acc_ref):
    i = pl.program_id(0)

    @pl.when(i == 0)
    def _init():
        q = q_ref[...]
        qn_ref[...] = (
            q / jnp.maximum(
                jnp.sqrt(jnp.sum(q * q, axis=1, keepdims=True)), _EPS
            )
        ).astype(jnp.bfloat16)
        acc_ref[...] = jnp.zeros_like(acc_ref)
        gam = gam_ref[...]
        eg = jnp.exp(gam - jnp.max(gam))
        ng_ref[...] = eg / jnp.sum(eg)

    g = g_ref[...]
    sumsq = jnp.sum(g * g, axis=1, keepdims=True)
    scale = _LOG2E * jax.lax.rsqrt(jnp.maximum(sumsq, _EPS * _EPS))
    gs = (g * scale).astype(jnp.bfloat16)
    sim2 = jax.lax.dot_general(
        qn_ref[...], gs,
        (((1,), (1,)), ((), ())),
        preferred_element_type=jnp.float32,
    )
    e = jnp.exp2(sim2.astype(jnp.bfloat16)).astype(_F8)
    gx = jnp.concatenate(
        [g, jnp.ones((_BLK, 1), jnp.float32)], axis=1
    ).astype(_F8)
    acc_ref[...] += jnp.dot(e, gx, preferred_element_type=jnp.float32)

    @pl.when(i == _NBLK - 1)
    def _fin():
        out_ref[...] = acc_ref[:, :_D] / acc_ref[:, _D:]


@jax.jit
def kernel(query, G, gamma):
    g_out, ng = pl.pallas_call(
        _attn_body,
        grid=(_NBLK,),
        in_specs=[
            pl.BlockSpec((_B, _D), lambda i: (0, 0)),
            pl.BlockSpec((100, 1000), lambda i: (0, 0)),
            pl.BlockSpec((_BLK, _D), lambda i: (i, 0)),
        ],
        out_specs=[
            pl.BlockSpec((_B, _D), lambda i: (0, 0)),
            pl.BlockSpec((100, 1000), lambda i: (0, 0)),
        ],
        out_shape=[
            jax.ShapeDtypeStruct((_B, _D), jnp.float32),
            jax.ShapeDtypeStruct((100, 1000), jnp.float32),
        ],
        scratch_shapes=[
            pltpu.VMEM((_B, _D), jnp.bfloat16),
            pltpu.VMEM((_B, _D + 1), jnp.float32),
        ],
    )(query, gamma.reshape(100, 1000), G)
    return g_out, ng.reshape(_N)


# speedup vs baseline: 3.0433x; 1.0479x over previous
import jax
import jax.numpy as jnp
from jax.experimental import pallas as pl
from jax.experimental.pallas import tpu as pltpu

_EPS = 1e-12
_LOG2E = 1.4426950408889634

_B = 1024
_D = 16
_N = 100000
_BLK = 5000
_NBLK = _N // _BLK
_F8 = jnp.float8_e4m3fn


def _attn_body(q_ref, gam_ref, g_ref, out_ref, ng_ref, qn_ref, acc_ref):
    i = pl.program_id(0)

    @pl.when(i == 0)
    def _init():
        q = q_ref[...]
        qn_ref[...] = (
            q / jnp.maximum(
                jnp.sqrt(jnp.sum(q * q, axis=1, keepdims=True)), _EPS
            )
        ).astype(jnp.bfloat16)
        acc_ref[...] = jnp.zeros_like(acc_ref)
        gam = gam_ref[...]
        eg = jnp.exp(gam - jnp.max(gam))
        ng_ref[...] = eg / jnp.sum(eg)

    g = g_ref[...]
    gt = g.T
    sumsq = jnp.sum(gt * gt, axis=0, keepdims=True)
    scale = _LOG2E * jax.lax.rsqrt(jnp.maximum(sumsq, _EPS * _EPS))
    gts = (gt * scale).astype(jnp.bfloat16)
    sim2 = jax.lax.dot_general(
        qn_ref[...], gts,
        (((1,), (0,)), ((), ())),
        preferred_element_type=jnp.float32,
    )
    e = jnp.exp2(sim2.astype(jnp.bfloat16)).astype(_F8)
    gx = jnp.concatenate(
        [g, jnp.ones((_BLK, 1), jnp.float32)], axis=1
    ).astype(_F8)
    acc_ref[...] += jnp.dot(e, gx, preferred_element_type=jnp.float32)

    @pl.when(i == _NBLK - 1)
    def _fin():
        out_ref[...] = acc_ref[:, :_D] / acc_ref[:, _D:]


@jax.jit
def kernel(query, G, gamma):
    g_out, ng = pl.pallas_call(
        _attn_body,
        grid=(_NBLK,),
        in_specs=[
            pl.BlockSpec((_B, _D), lambda i: (0, 0)),
            pl.BlockSpec((100, 1000), lambda i: (0, 0)),
            pl.BlockSpec((_BLK, _D), lambda i: (i, 0)),
        ],
        out_specs=[
            pl.BlockSpec((_B, _D), lambda i: (0, 0)),
            pl.BlockSpec((100, 1000), lambda i: (0, 0)),
        ],
        out_shape=[
            jax.ShapeDtypeStruct((_B, _D), jnp.float32),
            jax.ShapeDtypeStruct((100, 1000), jnp.float32),
        ],
        scratch_shapes=[
            pltpu.VMEM((_B, _D), jnp.bfloat16),
            pltpu.VMEM((_B, _D + 1), jnp.float32),
        ],
    )(query, gamma.reshape(100, 1000), G)
    return g_out, ng.reshape(_N)


# speedup vs baseline: 3.1178x; 1.0245x over previous
import jax
import jax.numpy as jnp
from jax.experimental import pallas as pl
from jax.experimental.pallas import tpu as pltpu

_EPS = 1e-12
_LOG2E = 1.4426950408889634

_B = 1024
_D = 16
_N = 100000
_BLK = 10000
_NBLK = _N // _BLK
_F8 = jnp.float8_e4m3fn


def _attn_body(q_ref, gam_ref, g_ref, out_ref, ng_ref, qn_ref, acc_ref):
    i = pl.program_id(0)

    @pl.when(i == 0)
    def _init():
        q = q_ref[...]
        qn_ref[...] = (
            q / jnp.maximum(
                jnp.sqrt(jnp.sum(q * q, axis=1, keepdims=True)), _EPS
            )
        ).astype(jnp.bfloat16)
        acc_ref[...] = jnp.zeros_like(acc_ref)
        gam = gam_ref[...]
        eg = jnp.exp(gam - jnp.max(gam))
        ng_ref[...] = eg / jnp.sum(eg)

    g = g_ref[...]
    gt = g.T
    sumsq = jnp.sum(gt * gt, axis=0, keepdims=True)
    scale = _LOG2E * jax.lax.rsqrt(jnp.maximum(sumsq, _EPS * _EPS))
    gts = (gt * scale).astype(jnp.bfloat16)
    sim2 = jax.lax.dot_general(
        qn_ref[...], gts,
        (((1,), (0,)), ((), ())),
        preferred_element_type=jnp.float32,
    )
    e = jnp.exp2(sim2.astype(jnp.bfloat16)).astype(_F8)
    gx = jnp.concatenate(
        [g, jnp.ones((_BLK, 1), jnp.float32)], axis=1
    ).astype(_F8)
    acc_ref[...] += jnp.dot(e, gx, preferred_element_type=jnp.float32)

    @pl.when(i == _NBLK - 1)
    def _fin():
        out_ref[...] = acc_ref[:, :_D] / acc_ref[:, _D:]


@jax.jit
def kernel(query, G, gamma):
    g_out, ng = pl.pallas_call(
        _attn_body,
        grid=(_NBLK,),
        in_specs=[
            pl.BlockSpec((_B, _D), lambda i: (0, 0)),
            pl.BlockSpec((100, 1000), lambda i: (0, 0)),
            pl.BlockSpec((_BLK, _D), lambda i: (i, 0)),
        ],
        out_specs=[
            pl.BlockSpec((_B, _D), lambda i: (0, 0)),
            pl.BlockSpec((100, 1000), lambda i: (0, 0)),
        ],
        out_shape=[
            jax.ShapeDtypeStruct((_B, _D), jnp.float32),
            jax.ShapeDtypeStruct((100, 1000), jnp.float32),
        ],
        scratch_shapes=[
            pltpu.VMEM((_B, _D), jnp.bfloat16),
            pltpu.VMEM((_B, _D + 1), jnp.float32),
        ],
    )(query, gamma.reshape(100, 1000), G)
    return g_out, ng.reshape(_N)
